# baseline (speedup 1.0000x reference)
import jax
import jax.numpy as jnp
from jax.experimental import pallas as pl
from jax.experimental.pallas import tpu as pltpu

T = 8192
D = 2048
DFF = 8192
E = 8
R = 16
ER = E * R

BT = 1024
BF = 1024
CH = 4
BC = BF // CH


def _mlp_body(x_ref, w1_ref, w2_ref, b1_ref, out_ref, g_ref):
    j = pl.program_id(1)
    x = x_ref[...]
    for k in range(CH):
        sl = slice(k * BC, (k + 1) * BC)
        h = jnp.dot(x, w1_ref[:, sl], preferred_element_type=jnp.float32)
        g_ref[:, sl] = jax.nn.gelu(h + b1_ref[:, sl]).astype(jnp.bfloat16)
    p = jnp.dot(g_ref[...], w2_ref[...],
                preferred_element_type=jnp.float32)

    @pl.when(j == 0)
    def _init():
        out_ref[...] = p

    @pl.when(j != 0)
    def _acc():
        out_ref[...] += p


def _mlp_partial(xb, w1b, w2b, b1r):
    t_tot = xb.shape[0]
    dff_loc = w1b.shape[1]
    grid = (t_tot // BT, dff_loc // BF)
    return pl.pallas_call(
        _mlp_body,
        grid=grid,
        in_specs=[
            pl.BlockSpec((BT, D), lambda i, j: (i, 0)),
            pl.BlockSpec((D, BF), lambda i, j: (0, j)),
            pl.BlockSpec((BF, D), lambda i, j: (j, 0)),
            pl.BlockSpec((1, BF), lambda i, j: (0, j)),
        ],
        out_specs=pl.BlockSpec((BT, D), lambda i, j: (i, 0)),
        out_shape=jax.ShapeDtypeStruct((t_tot, D), jnp.float32),
        scratch_shapes=[pltpu.VMEM((BT, BF), jnp.bfloat16)],
        compiler_params=pltpu.CompilerParams(
            dimension_semantics=("parallel", "arbitrary"),
        ),
    )(xb, w1b, w2b, b1r)


def _adapter_body(x_ref, mlp_ref, b2_ref, wr_ref, br_ref, a_ref, b_ref,
                  alpha_ref, out_ref):
    x = x_ref[...]
    le = jnp.dot(x, wr_ref[...], preferred_element_type=jnp.float32)
    le = le + br_ref[...]
    ex = jnp.exp(le - jnp.max(le, axis=-1, keepdims=True))
    eidx = jax.lax.broadcasted_iota(jnp.int32, ex.shape, 1) // R
    v1 = jnp.max(ex, axis=-1, keepdims=True)
    i1 = jnp.min(jnp.where(ex == v1, eidx, E), axis=-1, keepdims=True)
    m1 = eidx == i1
    ex2 = jnp.where(m1, -1.0, ex)
    v2 = jnp.max(ex2, axis=-1, keepdims=True)
    i2 = jnp.min(jnp.where(ex2 == v2, eidx, E), axis=-1, keepdims=True)
    m2 = eidx == i2
    combine = (jnp.where(m1, v1, 0.0) + jnp.where(m2, v2, 0.0)) / (v1 + v2)
    ha = jax.nn.gelu(jnp.dot(x, a_ref[...],
                             preferred_element_type=jnp.float32))
    hg = (ha * combine).astype(jnp.bfloat16)
    moe = jnp.dot(hg, b_ref[...], preferred_element_type=jnp.float32)
    out_ref[...] = mlp_ref[...] + b2_ref[...] + alpha_ref[0, 0] * moe


def _adapter(x_loc, mlp_loc, b2r, wr_exp, br_exp, a2d, b2d, alpha2d):
    t_loc = x_loc.shape[0]
    grid = (t_loc // BT,)
    return pl.pallas_call(
        _adapter_body,
        grid=grid,
        in_specs=[
            pl.BlockSpec((BT, D), lambda i: (i, 0)),
            pl.BlockSpec((BT, D), lambda i: (i, 0)),
            pl.BlockSpec((1, D), lambda i: (0, 0)),
            pl.BlockSpec((D, ER), lambda i: (0, 0)),
            pl.BlockSpec((1, ER), lambda i: (0, 0)),
            pl.BlockSpec((D, ER), lambda i: (0, 0)),
            pl.BlockSpec((ER, D), lambda i: (0, 0)),
            pl.BlockSpec((1, 1), lambda i: (0, 0)),
        ],
        out_specs=pl.BlockSpec((BT, D), lambda i: (i, 0)),
        out_shape=jax.ShapeDtypeStruct((t_loc, D), jnp.float32),
        compiler_params=pltpu.CompilerParams(
            dimension_semantics=("arbitrary",),
        ),
    )(x_loc, mlp_loc, b2r, wr_exp, br_exp, a2d, b2d, alpha2d)


def _run_tp(xb, w1h, w2h, b1h, b2r, wr_exp, br_exp, a2d, b2d, alpha2d):
    partial = _mlp_partial(xb, w1h, w2h, b1h)
    mlp_loc = jax.lax.psum_scatter(partial, "tp", scatter_dimension=0,
                                   tiled=True)
    t_loc = mlp_loc.shape[0]
    d_idx = jax.lax.axis_index("tp")
    x_loc = jax.lax.dynamic_slice(xb, (d_idx * t_loc, 0), (t_loc, D))
    return _adapter(x_loc, mlp_loc, b2r, wr_exp, br_exp, a2d, b2d, alpha2d)


def _run_single(xb, w1b, w2b, b1r, b2r, wr_exp, br_exp, a2d, b2d, alpha2d):
    partial = _mlp_partial(xb, w1b, w2b, b1r)
    return _adapter(xb, partial, b2r, wr_exp, br_exp, a2d, b2d, alpha2d)


def kernel(x, W1, b1, W2, b2, Wr, br, A, B, alpha):
    xb = x.astype(jnp.bfloat16)
    w1b = W1.astype(jnp.bfloat16)
    w2b = W2.astype(jnp.bfloat16)
    wr_exp = jnp.repeat(Wr, R, axis=1).astype(jnp.bfloat16)
    br_exp = jnp.repeat(br, R).reshape(1, ER)
    a2d = A.transpose(1, 0, 2).reshape(D, ER).astype(jnp.bfloat16)
    b2d = B.reshape(ER, D).astype(jnp.bfloat16)
    b1r = b1.reshape(1, DFF)
    b2r = b2.reshape(1, D)
    alpha2d = alpha.reshape(1, 1)

    devs = jax.devices()
    ndev = 2 if len(devs) >= 2 and T % (2 * BT) == 0 else 1
    if ndev == 1:
        return _run_single(xb, w1b, w2b, b1r, b2r, wr_exp, br_exp,
                           a2d, b2d, alpha2d)
    mesh = jax.sharding.Mesh(devs[:ndev], ("tp",))
    pspec = jax.sharding.PartitionSpec
    rep = pspec()
    fn = jax.shard_map(
        _run_tp,
        mesh=mesh,
        in_specs=(rep, pspec(None, "tp"), pspec("tp", None),
                  pspec(None, "tp"), rep, rep, rep, rep, rep, rep),
        out_specs=pspec("tp", None),
        check_vma=False,
    )
    return fn(xb, w1b, w2b, b1r, b2r, wr_exp, br_exp, a2d, b2d, alpha2d)


# speedup vs baseline: 1.4380x; 1.4380x over previous
import jax
import jax.numpy as jnp
from jax.experimental import pallas as pl
from jax.experimental.pallas import tpu as pltpu

T = 8192
D = 2048
DFF = 8192
E = 8
R = 16
ER = E * R

BT = 512
BF = 2048
CH = 8
BC = BF // CH


def _fused(x_ref, w1_ref, w2_ref, b1_ref, b2_ref, wr_ref, br_ref,
           a_ref, b_ref, alpha_ref, out_ref, g_ref):
    j = pl.program_id(1)
    x = x_ref[...]
    for k in range(CH):
        sl = slice(k * BC, (k + 1) * BC)
        h = jnp.dot(x, w1_ref[:, sl], preferred_element_type=jnp.float32)
        g_ref[:, sl] = jax.nn.gelu(h + b1_ref[:, sl]).astype(jnp.bfloat16)
    p = jnp.dot(g_ref[...], w2_ref[...],
                preferred_element_type=jnp.float32)

    @pl.when(j == 0)
    def _first():
        le = jnp.dot(x, wr_ref[...], preferred_element_type=jnp.float32)
        le = le + br_ref[...]
        ex = jnp.exp(le - jnp.max(le, axis=-1, keepdims=True))
        eidx = jax.lax.broadcasted_iota(jnp.int32, ex.shape, 1) // R
        v1 = jnp.max(ex, axis=-1, keepdims=True)
        i1 = jnp.min(jnp.where(ex == v1, eidx, E), axis=-1, keepdims=True)
        m1 = eidx == i1
        ex2 = jnp.where(m1, -1.0, ex)
        v2 = jnp.max(ex2, axis=-1, keepdims=True)
        i2 = jnp.min(jnp.where(ex2 == v2, eidx, E), axis=-1, keepdims=True)
        m2 = eidx == i2
        combine = (jnp.where(m1, v1, 0.0) + jnp.where(m2, v2, 0.0)) / (v1 + v2)
        ha = jax.nn.gelu(jnp.dot(x, a_ref[...],
                                 preferred_element_type=jnp.float32))
        hg = (ha * combine).astype(jnp.bfloat16)
        moe = jnp.dot(hg, b_ref[...], preferred_element_type=jnp.float32)
        out_ref[...] = p + b2_ref[...] + alpha_ref[0, 0] * moe

    @pl.when(j != 0)
    def _rest():
        out_ref[...] += p


def kernel(x, W1, b1, W2, b2, Wr, br, A, B, alpha):
    xb = x.astype(jnp.bfloat16)
    w1b = W1.astype(jnp.bfloat16)
    w2b = W2.astype(jnp.bfloat16)
    wr_exp = jnp.repeat(Wr, R, axis=1).astype(jnp.bfloat16)
    br_exp = jnp.repeat(br, R).reshape(1, ER)
    a2d = A.transpose(1, 0, 2).reshape(D, ER).astype(jnp.bfloat16)
    b2d = B.reshape(ER, D).astype(jnp.bfloat16)
    b1r = b1.reshape(1, DFF)
    b2r = b2.reshape(1, D)
    alpha2d = alpha.reshape(1, 1)

    grid = (T // BT, DFF // BF)
    return pl.pallas_call(
        _fused,
        grid=grid,
        in_specs=[
            pl.BlockSpec((BT, D), lambda i, j: (i, 0)),
            pl.BlockSpec((D, BF), lambda i, j: (0, j)),
            pl.BlockSpec((BF, D), lambda i, j: (j, 0)),
            pl.BlockSpec((1, BF), lambda i, j: (0, j)),
            pl.BlockSpec((1, D), lambda i, j: (0, 0)),
            pl.BlockSpec((D, ER), lambda i, j: (0, 0)),
            pl.BlockSpec((1, ER), lambda i, j: (0, 0)),
            pl.BlockSpec((D, ER), lambda i, j: (0, 0)),
            pl.BlockSpec((ER, D), lambda i, j: (0, 0)),
            pl.BlockSpec((1, 1), lambda i, j: (0, 0)),
        ],
        out_specs=pl.BlockSpec((BT, D), lambda i, j: (i, 0)),
        out_shape=jax.ShapeDtypeStruct((T, D), jnp.float32),
        scratch_shapes=[pltpu.VMEM((BT, BF), jnp.bfloat16)],
        compiler_params=pltpu.CompilerParams(
            dimension_semantics=("parallel", "arbitrary"),
        ),
    )(xb, w1b, w2b, b1r, b2r, wr_exp, br_exp, a2d, b2d, alpha2d)
